# Initial kernel scaffold; baseline (speedup 1.0000x reference)
import jax
import jax.numpy as jnp
from jax.experimental import pallas as pl


def kernel(g, h, pre_h, idx):
    raise NotImplementedError("write your pallas kernel here")


# speedup vs baseline: 4.0766x; 4.0766x over previous
import functools

import jax
import jax.numpy as jnp
from jax.experimental import pallas as pl

B, H, N_SMALL, N, D = 8, 4, 1024, 2048, 128
BN = 256


def _unpool_body(idx_ref, h_ref, pre_ref, out_ref):
    nb = pl.program_id(2)
    n0 = nb * BN
    idx_row = idx_ref[...].reshape(1, N_SMALL)
    tids = n0 + jax.lax.broadcasted_iota(jnp.int32, (BN, 1), 0)
    p = (idx_row == tids).astype(jnp.float32)
    hmat = h_ref[...].reshape(N_SMALL, D)
    acc = jnp.dot(p, hmat, preferred_element_type=jnp.float32)
    out_ref[...] = (acc + pre_ref[...].reshape(BN, D)).reshape(1, 1, BN, D)


def _unpool(h, pre_h, idx32):
    grid = (B, H, N // BN)
    return pl.pallas_call(
        _unpool_body,
        grid=grid,
        in_specs=[
            pl.BlockSpec((1, 1, N_SMALL), lambda b, hh, nb: (b, 0, 0)),
            pl.BlockSpec((1, 1, N_SMALL, D), lambda b, hh, nb: (b, hh, 0, 0)),
            pl.BlockSpec((1, 1, BN, D), lambda b, hh, nb: (b, hh, nb, 0)),
        ],
        out_specs=pl.BlockSpec((1, 1, BN, D), lambda b, hh, nb: (b, hh, nb, 0)),
        out_shape=jax.ShapeDtypeStruct((B, H, N, D), jnp.float32),
    )(idx32.reshape(B, 1, N_SMALL), h, pre_h)


def kernel(g, h, pre_h, idx):
    idx32 = idx.astype(jnp.int32)
    new_h = _unpool(h, pre_h, idx32)
    return (g, new_h)


# speedup vs baseline: 6.3825x; 1.5657x over previous
import functools

import jax
import jax.numpy as jnp
from jax import lax
from jax.experimental import pallas as pl
from jax.experimental.pallas import tpu as pltpu
from jax.experimental.pallas import tpu_sc as plsc

B, H, N_SMALL, N, D = 8, 4, 1024, 2048, 128
C = 256
NCHUNK_A = N // C
NCHUNK_B = N_SMALL // C
SUB = 128


def _sc_unpool(h, pre_h, idx3):
    mesh = plsc.VectorSubcoreMesh(core_axis_name="c", subcore_axis_name="s")

    @functools.partial(
        pl.kernel,
        mesh=mesh,
        out_type=jax.ShapeDtypeStruct((B, H, N, D), jnp.float32),
        scratch_types=[
            pltpu.VMEM((C, D), jnp.float32),
            pltpu.VMEM((C, D), jnp.float32),
            pltpu.VMEM((C // SUB, SUB), jnp.int32),
            pltpu.SemaphoreType.DMA,
        ],
    )
    def k(h_hbm, pre_hbm, idx_hbm, out_hbm, pbuf, hbuf, idxv, sem):
        cid = lax.axis_index("c")
        sid = lax.axis_index("s")
        wid = sid * 2 + cid
        b = wid // H
        hh = wid % H

        def body_a(ca, _):
            n0 = ca * C
            pltpu.sync_copy(pre_hbm.at[b, hh, pl.ds(n0, C)], pbuf)
            pltpu.sync_copy(pbuf, out_hbm.at[b, hh, pl.ds(n0, C)])
            return _
        lax.fori_loop(0, NCHUNK_A, body_a, 0, unroll=False)

        def body_b(cb, _):
            j0 = cb * C
            pltpu.sync_copy(idx_hbm.at[b, pl.ds(cb * (C // SUB), C // SUB)],
                            idxv)
            pltpu.sync_copy(h_hbm.at[b, hh, pl.ds(j0, C)], hbuf)
            for s in range(C // SUB):
                pltpu.async_copy(
                    pre_hbm.at[b, hh].at[idxv.at[s]],
                    pbuf.at[pl.ds(s * SUB, SUB)],
                    sem,
                ).wait()

            def addrow(r, _):
                for l in range(D // 16):
                    sl = pl.ds(l * 16, 16)
                    plsc.addupdate(pbuf.at[r, sl], hbuf[r, sl])
                return _
            lax.fori_loop(0, C, addrow, 0, unroll=False)

            for s in range(C // SUB):
                pltpu.async_copy(
                    pbuf.at[pl.ds(s * SUB, SUB)],
                    out_hbm.at[b, hh].at[idxv.at[s]],
                    sem,
                ).wait()
            return _
        lax.fori_loop(0, NCHUNK_B, body_b, 0, unroll=False)

    return k(h, pre_h, idx3)


def kernel(g, h, pre_h, idx):
    idx3 = idx.astype(jnp.int32).reshape(B, N_SMALL // SUB, SUB)
    new_h = _sc_unpool(h, pre_h, idx3)
    return (g, new_h)
